# baseline (device time: 14208 ns/iter reference)
import jax
import jax.numpy as jnp
from jax import lax
from jax.experimental import pallas as pl
from jax.experimental.pallas import tpu as pltpu

N_DEV = 4


def kernel(x, W, labels):
    T, D = x.shape
    _, V = W.shape

    def body(x_ref, w_ref, lab_ref, out_ref, comm_ref, send_sems, recv_sems):
        my_pos = lax.axis_index("i")
        partner_a = jnp.bitwise_xor(my_pos, 1)
        partner_b = 3 - my_pos
        barrier_sem = pltpu.get_barrier_semaphore()
        for peer in (partner_a, partner_b):
            pl.semaphore_signal(
                barrier_sem, inc=1,
                device_id=(peer,), device_id_type=pl.DeviceIdType.MESH,
            )

        xb = x_ref[...].astype(jnp.bfloat16)
        wb = w_ref[...].astype(jnp.bfloat16)
        logits_t = lax.dot_general(
            wb, xb,
            dimension_numbers=(((0,), (1,)), ((), ())),
            preferred_element_type=jnp.float32,
        ).astype(jnp.bfloat16)

        e_t = jnp.exp(logits_t)
        vio = lax.broadcasted_iota(jnp.int32, (V, T), 0) + my_pos * V
        lab_row = lab_ref[...].reshape(1, T)
        masked_t = jnp.where(vio == lab_row, logits_t, jnp.bfloat16(0.0))

        ones8 = jnp.ones((8, V), jnp.bfloat16)
        s8 = lax.dot_general(
            ones8, e_t, dimension_numbers=(((1,), (0,)), ((), ())),
            preferred_element_type=jnp.float32,
        )
        c8 = lax.dot_general(
            ones8, masked_t, dimension_numbers=(((1,), (0,)), ((), ())),
            preferred_element_type=jnp.float32,
        )
        comm_ref[0] = jnp.concatenate([s8[0:1], c8[0:1]], axis=0)

        pl.semaphore_wait(barrier_sem, 2)

        rdma_a = pltpu.make_async_remote_copy(
            src_ref=comm_ref.at[0],
            dst_ref=comm_ref.at[1],
            send_sem=send_sems.at[0],
            recv_sem=recv_sems.at[0],
            device_id=(partner_a,),
            device_id_type=pl.DeviceIdType.MESH,
        )
        rdma_a.start()
        rdma_a.wait_recv()
        half = comm_ref[0] + comm_ref[1]
        rdma_a.wait_send()

        comm_ref[0] = half
        rdma_b = pltpu.make_async_remote_copy(
            src_ref=comm_ref.at[0],
            dst_ref=comm_ref.at[2],
            send_sem=send_sems.at[1],
            recv_sem=recv_sems.at[1],
            device_id=(partner_b,),
            device_id_type=pl.DeviceIdType.MESH,
        )
        rdma_b.start()
        rdma_b.wait_recv()

        tot = comm_ref[0] + comm_ref[2]
        out_ref[...] = (jnp.log(tot[0:1]) - tot[1:2]).reshape(T)

        rdma_b.wait_send()

    out = pl.pallas_call(
        body,
        out_shape=jax.ShapeDtypeStruct((T,), jnp.float32),
        in_specs=[
            pl.BlockSpec(memory_space=pltpu.MemorySpace.VMEM),
            pl.BlockSpec(memory_space=pltpu.MemorySpace.VMEM),
            pl.BlockSpec(memory_space=pltpu.MemorySpace.VMEM),
        ],
        out_specs=pl.BlockSpec(memory_space=pltpu.MemorySpace.VMEM),
        scratch_shapes=[
            pltpu.VMEM((3, 2, T), jnp.float32),
            pltpu.SemaphoreType.DMA((2,)),
            pltpu.SemaphoreType.DMA((2,)),
        ],
        compiler_params=pltpu.CompilerParams(collective_id=0),
    )(x, W, labels)
    return out


# device time: 11518 ns/iter; 1.2335x vs baseline; 1.2335x over previous
import jax
import jax.numpy as jnp
from jax import lax
from jax.experimental import pallas as pl
from jax.experimental.pallas import tpu as pltpu

N_DEV = 4


def kernel(x, W, labels):
    T, D = x.shape
    _, V = W.shape

    def body(x_hbm, w_hbm, lab_hbm, out_ref,
             x_ref, w_ref, lab_ref, comm_ref,
             in_sems, send_sems, recv_sems):
        my_pos = lax.axis_index("i")
        barrier_sem = pltpu.get_barrier_semaphore()
        for j in range(1, N_DEV):
            peer = lax.rem(my_pos + j, N_DEV)
            pl.semaphore_signal(
                barrier_sem, inc=1,
                device_id=(peer,), device_id_type=pl.DeviceIdType.MESH,
            )

        cw = pltpu.make_async_copy(w_hbm, w_ref, in_sems.at[0])
        cx = pltpu.make_async_copy(x_hbm, x_ref, in_sems.at[1])
        cl = pltpu.make_async_copy(lab_hbm, lab_ref, in_sems.at[2])
        cw.start()
        cx.start()
        cl.start()
        cx.wait()
        cl.wait()
        cw.wait()

        xb = x_ref[...].astype(jnp.bfloat16)
        wb = w_ref[...].astype(jnp.bfloat16)
        logits_t = lax.dot_general(
            wb, xb,
            dimension_numbers=(((0,), (1,)), ((), ())),
            preferred_element_type=jnp.float32,
        ).astype(jnp.bfloat16)

        e_t = jnp.exp(logits_t)
        vio = lax.broadcasted_iota(jnp.int32, (V, T), 0) + my_pos * V
        lab_row = lab_ref[...].reshape(1, T)
        masked_t = jnp.where(vio == lab_row, logits_t, jnp.bfloat16(0.0))

        ones8 = jnp.ones((8, V), jnp.bfloat16)
        s8 = lax.dot_general(
            ones8, e_t, dimension_numbers=(((1,), (0,)), ((), ())),
            preferred_element_type=jnp.float32,
        )
        c8 = lax.dot_general(
            ones8, masked_t, dimension_numbers=(((1,), (0,)), ((), ())),
            preferred_element_type=jnp.float32,
        )
        comm_ref[0] = jnp.concatenate([s8[0:1], c8[0:1]], axis=0)

        pl.semaphore_wait(barrier_sem, N_DEV - 1)

        rdmas = []
        for j in range(1, N_DEV):
            peer = lax.rem(my_pos + j, N_DEV)
            rdma = pltpu.make_async_remote_copy(
                src_ref=comm_ref.at[0],
                dst_ref=comm_ref.at[j],
                send_sem=send_sems.at[j - 1],
                recv_sem=recv_sems.at[j - 1],
                device_id=(peer,),
                device_id_type=pl.DeviceIdType.MESH,
            )
            rdma.start()
            rdmas.append(rdma)
        for rdma in rdmas:
            rdma.wait_recv()

        tot = comm_ref[0] + comm_ref[1] + comm_ref[2] + comm_ref[3]
        out_ref[...] = (jnp.log(tot[0:1]) - tot[1:2]).reshape(T)

        for rdma in rdmas:
            rdma.wait_send()

    out = pl.pallas_call(
        body,
        out_shape=jax.ShapeDtypeStruct((T,), jnp.float32),
        in_specs=[
            pl.BlockSpec(memory_space=pltpu.MemorySpace.HBM),
            pl.BlockSpec(memory_space=pltpu.MemorySpace.HBM),
            pl.BlockSpec(memory_space=pltpu.MemorySpace.HBM),
        ],
        out_specs=pl.BlockSpec(memory_space=pltpu.MemorySpace.VMEM),
        scratch_shapes=[
            pltpu.VMEM((T, D), jnp.float32),
            pltpu.VMEM((D, V), jnp.float32),
            pltpu.VMEM((T,), jnp.int32),
            pltpu.VMEM((N_DEV, 2, T), jnp.float32),
            pltpu.SemaphoreType.DMA((3,)),
            pltpu.SemaphoreType.DMA((N_DEV - 1,)),
            pltpu.SemaphoreType.DMA((N_DEV - 1,)),
        ],
        compiler_params=pltpu.CompilerParams(collective_id=0),
    )(
        pltpu.with_memory_space_constraint(x, pltpu.MemorySpace.HBM),
        pltpu.with_memory_space_constraint(W, pltpu.MemorySpace.HBM),
        pltpu.with_memory_space_constraint(labels, pltpu.MemorySpace.HBM),
    )
    return out
